# Initial kernel scaffold; baseline (speedup 1.0000x reference)
import functools

import jax
import jax.numpy as jnp
from jax import lax
from jax.experimental import pallas as pl
from jax.experimental.pallas import tpu as pltpu
from jax.experimental.pallas import tpu_sc as plsc

_MAX_ENTITY = 5976
_GROUP_MAX = 12
_ENT_DIM = 102
_GRP_DIM = 26
_D = 128
_NUM_GLYPHS = 6000
_GP = 6144
_EP = 5984
_GTR = 16
_NC, _NS = 2, 16
_NW = _NC * _NS
_ACHUNK = 128


def _renorm_body(e_ref, g_ref, eo_ref, go_ref):
    for src, dst in ((e_ref, eo_ref), (g_ref, go_ref)):
        x = src[...]
        s = jnp.sum(x * x, axis=1, keepdims=True)
        scale = jnp.minimum(1.0, 1.0 / jnp.maximum(jnp.sqrt(s), 1e-7))
        dst[...] = x * scale


_renorm = pl.pallas_call(
    _renorm_body,
    out_shape=(
        jax.ShapeDtypeStruct((_EP, _D), jnp.float32),
        jax.ShapeDtypeStruct((_GTR, _D), jnp.float32),
    ),
)


def _sc_body(rw, cb, ern, grn, eidx, gidx, gly, out,
             gtab_v, eidx_v, gidx_v, cbuf, c_sp, gly_v, obuf, sem):
    cid = lax.axis_index("c")
    sid = lax.axis_index("s")
    lane = lax.iota(jnp.int32, 16)
    rows_a = _GP // _NS

    pltpu.sync_copy(grn, gtab_v)

    def chunk_a(j, carry):
        base = sid * rows_a + j * _ACHUNK
        pltpu.sync_copy(eidx.at[pl.ds(base, _ACHUNK)], eidx_v)
        pltpu.sync_copy(gidx.at[pl.ds(base, _ACHUNK)], gidx_v)
        pltpu.async_copy(ern.at[eidx_v], cbuf, sem).wait()

        def grp16(g, carry2):
            rvec = g * 16 + lane
            gvec = gidx_v[pl.ds(g * 16, 16)]

            def col(c, carry3):
                vals = plsc.load_gather(
                    gtab_v, [gvec, jnp.full((16,), 0, jnp.int32) + c])
                plsc.store_scatter(
                    cbuf,
                    [rvec, jnp.full((16,), _ENT_DIM, jnp.int32) + c],
                    vals)
                return carry3

            return lax.fori_loop(0, _GRP_DIM, col, carry2)

        lax.fori_loop(0, _ACHUNK // 16, grp16, 0)
        pltpu.sync_copy(cbuf, c_sp.at[pl.ds(base, _ACHUNK)])
        return carry

    lax.fori_loop(0, rows_a // _ACHUNK, chunk_a, 0)
    plsc.subcore_barrier()

    w = sid * _NC + cid

    def chunk_b(i, carry):
        b = w * rw + i * cb
        pltpu.sync_copy(gly.at[pl.ds(b, cb)], gly_v)
        pltpu.async_copy(c_sp.at[gly_v], obuf, sem).wait()
        pltpu.sync_copy(obuf, out.at[pl.ds(b, cb)])
        return carry

    lax.fori_loop(0, rw // cb, chunk_b, 0)


@functools.lru_cache(maxsize=None)
def _make_sc_call(n_rows):
    assert n_rows % _NW == 0
    rw = n_rows // _NW
    cb = next(d for d in range(128, 0, -8) if rw % d == 0)
    mesh = plsc.VectorSubcoreMesh(
        core_axis_name="c", subcore_axis_name="s",
        num_cores=_NC, num_subcores=_NS)
    return pl.kernel(
        functools.partial(_sc_body, rw, cb),
        out_type=jax.ShapeDtypeStruct((n_rows, _D), jnp.float32),
        mesh=mesh,
        scratch_types=[
            pltpu.VMEM((_GTR, _D), jnp.float32),
            pltpu.VMEM((_ACHUNK,), jnp.int32),
            pltpu.VMEM((_ACHUNK,), jnp.int32),
            pltpu.VMEM((_ACHUNK, _D), jnp.float32),
            pltpu.VMEM_SHARED((_GP, _D), jnp.float32),
            pltpu.VMEM((cb,), jnp.int32),
            pltpu.VMEM((cb, _D), jnp.float32),
            pltpu.SemaphoreType.DMA,
        ],
    )


def kernel(glyphs, gl_lookup, entity_table, group_table):
    ent_rn, grp_rn = _renorm(
        jnp.pad(entity_table,
                ((0, _EP - (_MAX_ENTITY + 1)), (0, _D - _ENT_DIM))),
        jnp.pad(group_table,
                ((0, _GTR - (_GROUP_MAX + 1)), (0, _D - _GRP_DIM))),
    )
    gl_grp = jnp.pad(gl_lookup[:, 0], (0, _GP - _NUM_GLYPHS))
    gl_ent = jnp.pad(gl_lookup[:, 1], (0, _GP - _NUM_GLYPHS))
    gly = glyphs.reshape(-1)
    out = _make_sc_call(gly.shape[0])(ent_rn, grp_rn, gl_ent, gl_grp, gly)
    return out.reshape(glyphs.shape + (_D,))


# speedup vs baseline: 6.2841x; 6.2841x over previous
import functools

import jax
import jax.numpy as jnp
from jax import lax
from jax.experimental import pallas as pl
from jax.experimental.pallas import tpu as pltpu
from jax.experimental.pallas import tpu_sc as plsc

_MAX_ENTITY = 5976
_GROUP_MAX = 12
_ENT_DIM = 102
_GRP_DIM = 26
_D = 128
_NUM_GLYPHS = 6000
_GP = 6144
_EP = 5984
_GTR = 16
_NC, _NS = 2, 16
_NW = _NC * _NS
_ACHUNK = 128


def _renorm_body(e_ref, g_ref, eo_ref, go_ref):
    x = e_ref[...]
    s = jnp.sum(x * x, axis=1, keepdims=True)
    scale = jnp.minimum(1.0, 1.0 / jnp.maximum(jnp.sqrt(s), 1e-7))
    eo_ref[...] = x * scale

    y = g_ref[...]
    s = jnp.sum(y * y, axis=1, keepdims=True)
    scale = jnp.minimum(1.0, 1.0 / jnp.maximum(jnp.sqrt(s), 1e-7))
    y = y * scale
    go_ref[...] = jnp.concatenate(
        [y[:, _GRP_DIM:], y[:, :_GRP_DIM]], axis=1)


_renorm = pl.pallas_call(
    _renorm_body,
    out_shape=(
        jax.ShapeDtypeStruct((_EP, _D), jnp.float32),
        jax.ShapeDtypeStruct((_GTR, _D), jnp.float32),
    ),
)


def _sc_body(rw, cb, ern, gsn, eidx, gidx, gly, out,
             eidx_v, gidx_v, cbuf, c_sp, gly_v, obuf, sem):
    cid = lax.axis_index("c")
    sid = lax.axis_index("s")
    rows_a = _GP // _NS

    def chunk_a(j, carry):
        base = sid * rows_a + j * _ACHUNK
        pltpu.sync_copy(eidx.at[pl.ds(base, _ACHUNK)], eidx_v)
        pltpu.sync_copy(gidx.at[pl.ds(base, _ACHUNK)], gidx_v)
        pltpu.async_copy(ern.at[eidx_v], cbuf, sem).wait()
        pltpu.async_copy(gsn.at[gidx_v], cbuf, sem, add=True).wait()
        pltpu.sync_copy(cbuf, c_sp.at[pl.ds(base, _ACHUNK)])
        return carry

    lax.fori_loop(0, rows_a // _ACHUNK, chunk_a, 0)
    plsc.subcore_barrier()

    w = sid * _NC + cid

    def chunk_b(i, carry):
        b = w * rw + i * cb
        pltpu.sync_copy(gly.at[pl.ds(b, cb)], gly_v)
        pltpu.async_copy(c_sp.at[gly_v], obuf, sem).wait()
        pltpu.sync_copy(obuf, out.at[pl.ds(b, cb)])
        return carry

    lax.fori_loop(0, rw // cb, chunk_b, 0)


@functools.lru_cache(maxsize=None)
def _make_sc_call(n_rows):
    assert n_rows % _NW == 0
    rw = n_rows // _NW
    cb = next(d for d in range(128, 0, -8) if rw % d == 0)
    mesh = plsc.VectorSubcoreMesh(
        core_axis_name="c", subcore_axis_name="s",
        num_cores=_NC, num_subcores=_NS)
    return pl.kernel(
        functools.partial(_sc_body, rw, cb),
        out_type=jax.ShapeDtypeStruct((n_rows, _D), jnp.float32),
        mesh=mesh,
        scratch_types=[
            pltpu.VMEM((_ACHUNK,), jnp.int32),
            pltpu.VMEM((_ACHUNK,), jnp.int32),
            pltpu.VMEM((_ACHUNK, _D), jnp.float32),
            pltpu.VMEM_SHARED((_GP, _D), jnp.float32),
            pltpu.VMEM((cb,), jnp.int32),
            pltpu.VMEM((cb, _D), jnp.float32),
            pltpu.SemaphoreType.DMA,
        ],
    )


def kernel(glyphs, gl_lookup, entity_table, group_table):
    ent_rn, grp_sh = _renorm(
        jnp.pad(entity_table,
                ((0, _EP - (_MAX_ENTITY + 1)), (0, _D - _ENT_DIM))),
        jnp.pad(group_table,
                ((0, _GTR - (_GROUP_MAX + 1)), (0, _D - _GRP_DIM))),
    )
    gl_grp = jnp.pad(gl_lookup[:, 0], (0, _GP - _NUM_GLYPHS))
    gl_ent = jnp.pad(gl_lookup[:, 1], (0, _GP - _NUM_GLYPHS))
    gly = glyphs.reshape(-1)
    out = _make_sc_call(gly.shape[0])(ent_rn, grp_sh, gl_ent, gl_grp, gly)
    return out.reshape(glyphs.shape + (_D,))


# speedup vs baseline: 7.9571x; 1.2662x over previous
import functools

import jax
import jax.numpy as jnp
from jax import lax
from jax.experimental import pallas as pl
from jax.experimental.pallas import tpu as pltpu
from jax.experimental.pallas import tpu_sc as plsc

_MAX_ENTITY = 5976
_GROUP_MAX = 12
_ENT_DIM = 102
_GRP_DIM = 26
_D = 128
_NUM_GLYPHS = 6000
_GP = 6144
_EP = 5984
_GTR = 16
_NC, _NS = 2, 16
_NW = _NC * _NS
_ACHUNK = 128


def _renorm_body(e_ref, g_ref, eo_ref, go_ref):
    x = e_ref[...]
    s = jnp.sum(x * x, axis=1, keepdims=True)
    scale = jnp.minimum(1.0, 1.0 / jnp.maximum(jnp.sqrt(s), 1e-7))
    eo_ref[...] = x * scale

    y = g_ref[...]
    s = jnp.sum(y * y, axis=1, keepdims=True)
    scale = jnp.minimum(1.0, 1.0 / jnp.maximum(jnp.sqrt(s), 1e-7))
    y = y * scale
    go_ref[...] = jnp.concatenate(
        [y[:, _GRP_DIM:], y[:, :_GRP_DIM]], axis=1)


_renorm = pl.pallas_call(
    _renorm_body,
    out_shape=(
        jax.ShapeDtypeStruct((_EP, _D), jnp.float32),
        jax.ShapeDtypeStruct((_GTR, _D), jnp.float32),
    ),
)


_NBUF = 3


def _sc_body(it, cb, ern, gsn, eidx, gidx, gly, out,
             eidx_v, gidx_v, cbuf, c_sp,
             ib0, ib1, ib2, ob0, ob1, ob2,
             si0, si1, si2, sg0, sg1, sg2, sw0, sw1, sw2):
    cid = lax.axis_index("c")
    sid = lax.axis_index("s")
    rows_a = _GP // _NS
    rw = it * cb
    ibs = (ib0, ib1, ib2)
    obs = (ob0, ob1, ob2)
    sis = (si0, si1, si2)
    sgs = (sg0, sg1, sg2)
    sws = (sw0, sw1, sw2)

    def chunk_a(j, carry):
        base = sid * rows_a + j * _ACHUNK
        pltpu.sync_copy(eidx.at[pl.ds(base, _ACHUNK)], eidx_v)
        pltpu.sync_copy(gidx.at[pl.ds(base, _ACHUNK)], gidx_v)
        pltpu.async_copy(ern.at[eidx_v], cbuf, sg0).wait()
        pltpu.async_copy(gsn.at[gidx_v], cbuf, sg0, add=True).wait()
        pltpu.sync_copy(cbuf, c_sp.at[pl.ds(base, _ACHUNK)])
        return carry

    lax.fori_loop(0, rows_a // _ACHUNK, chunk_a, 0)
    plsc.subcore_barrier()

    w = sid * _NC + cid

    def start_i(i, b):
        pltpu.async_copy(gly.at[pl.ds(w * rw + i * cb, cb)], ibs[b], sis[b])

    def wait_i(b):
        pltpu.make_async_copy(
            gly.at[pl.ds(w * rw, cb)], ibs[b], sis[b]).wait()

    def start_g(b):
        pltpu.async_copy(c_sp.at[ibs[b]], obs[b], sgs[b])

    def wait_g(b):
        pltpu.make_async_copy(c_sp.at[ibs[b]], obs[b], sgs[b]).wait()

    def start_w(i, b):
        pltpu.async_copy(obs[b], out.at[pl.ds(w * rw + i * cb, cb)], sws[b])

    def wait_w(b):
        pltpu.make_async_copy(
            obs[b], out.at[pl.ds(w * rw, cb)], sws[b]).wait()

    for b in range(_NBUF):
        start_i(b, b)
    for b in range(_NBUF):
        wait_i(b)
        start_g(b)

    def ring(k, carry):
        for b in range(_NBUF):
            i = k * _NBUF + b
            wait_g(b)
            start_w(i, b)

            @pl.when(k < it // _NBUF - 1)
            def _():
                start_i(i + _NBUF, b)
                wait_w(b)
                wait_i(b)
                start_g(b)

        return carry

    lax.fori_loop(0, it // _NBUF, ring, 0)
    for b in range(_NBUF):
        wait_w(b)


@functools.lru_cache(maxsize=None)
def _make_sc_call(n_rows):
    assert n_rows % _NW == 0
    rw = n_rows // _NW
    cb = next(d for d in range(128, 0, -8)
              if rw % d == 0 and (rw // d) % _NBUF == 0)
    it = rw // cb
    mesh = plsc.VectorSubcoreMesh(
        core_axis_name="c", subcore_axis_name="s",
        num_cores=_NC, num_subcores=_NS)
    call = pl.kernel(
        functools.partial(_sc_body, it, cb),
        out_type=jax.ShapeDtypeStruct((n_rows, _D), jnp.float32),
        mesh=mesh,
        scratch_types=[
            pltpu.VMEM((_ACHUNK,), jnp.int32),
            pltpu.VMEM((_ACHUNK,), jnp.int32),
            pltpu.VMEM((_ACHUNK, _D), jnp.float32),
            pltpu.VMEM_SHARED((_GP, _D), jnp.float32),
            pltpu.VMEM((cb,), jnp.int32),
            pltpu.VMEM((cb,), jnp.int32),
            pltpu.VMEM((cb,), jnp.int32),
            pltpu.VMEM((cb, _D), jnp.float32),
            pltpu.VMEM((cb, _D), jnp.float32),
            pltpu.VMEM((cb, _D), jnp.float32),
            pltpu.SemaphoreType.DMA,
            pltpu.SemaphoreType.DMA,
            pltpu.SemaphoreType.DMA,
            pltpu.SemaphoreType.DMA,
            pltpu.SemaphoreType.DMA,
            pltpu.SemaphoreType.DMA,
            pltpu.SemaphoreType.DMA,
            pltpu.SemaphoreType.DMA,
            pltpu.SemaphoreType.DMA,
        ],
    )
    return call


def kernel(glyphs, gl_lookup, entity_table, group_table):
    ent_rn, grp_sh = _renorm(
        jnp.pad(entity_table,
                ((0, _EP - (_MAX_ENTITY + 1)), (0, _D - _ENT_DIM))),
        jnp.pad(group_table,
                ((0, _GTR - (_GROUP_MAX + 1)), (0, _D - _GRP_DIM))),
    )
    gl_grp = jnp.pad(gl_lookup[:, 0], (0, _GP - _NUM_GLYPHS))
    gl_ent = jnp.pad(gl_lookup[:, 1], (0, _GP - _NUM_GLYPHS))
    gly = glyphs.reshape(-1)
    out = _make_sc_call(gly.shape[0])(ent_rn, grp_sh, gl_ent, gl_grp, gly)
    return out.reshape(glyphs.shape + (_D,))


# speedup vs baseline: 29.2443x; 3.6753x over previous
import functools

import jax
import jax.numpy as jnp
from jax import lax
from jax.experimental import pallas as pl
from jax.experimental.pallas import tpu as pltpu
from jax.experimental.pallas import tpu_sc as plsc

_MAX_ENTITY = 5976
_GROUP_MAX = 12
_ENT_DIM = 102
_GRP_DIM = 26
_D = 128
_NUM_GLYPHS = 6000
_GP = 6144
_EP = 5984
_GTR = 16
_NC, _NS = 2, 16
_NW = _NC * _NS
_ACHUNK = 128


def _renorm_body(e_ref, g_ref, eo_ref, go_ref):
    x = e_ref[...]
    s = jnp.sum(x * x, axis=1, keepdims=True)
    scale = jnp.minimum(1.0, 1.0 / jnp.maximum(jnp.sqrt(s), 1e-7))
    eo_ref[...] = x * scale

    y = g_ref[...]
    s = jnp.sum(y * y, axis=1, keepdims=True)
    scale = jnp.minimum(1.0, 1.0 / jnp.maximum(jnp.sqrt(s), 1e-7))
    y = y * scale
    go_ref[...] = jnp.concatenate(
        [y[:, _GRP_DIM:], y[:, :_GRP_DIM]], axis=1)


_renorm = pl.pallas_call(
    _renorm_body,
    out_shape=(
        jax.ShapeDtypeStruct((_EP, _D), jnp.float32),
        jax.ShapeDtypeStruct((_GTR, _D), jnp.float32),
    ),
)


_NBUF = 3


def _sc_body(it, cb, ern, gsn, eidx, gidx, gly, out,
             eidx_v, gidx_v, cbuf, c_sp,
             ib0, ib1, ib2, ob0, ob1, ob2,
             si0, si1, si2, sg0, sg1, sg2, sw0, sw1, sw2):
    cid = lax.axis_index("c")
    sid = lax.axis_index("s")
    rows_a = _GP // _NS
    rw = it * cb
    ibs = (ib0, ib1, ib2)
    obs = (ob0, ob1, ob2)
    sis = (si0, si1, si2)
    sgs = (sg0, sg1, sg2)
    sws = (sw0, sw1, sw2)

    def chunk_a(j, carry):
        base = sid * rows_a + j * _ACHUNK
        pltpu.sync_copy(eidx.at[pl.ds(base, _ACHUNK)], eidx_v)
        pltpu.sync_copy(gidx.at[pl.ds(base, _ACHUNK)], gidx_v)
        pltpu.async_copy(ern.at[eidx_v], cbuf, sg0).wait()
        pltpu.async_copy(gsn.at[gidx_v], cbuf, sg0, add=True).wait()
        pltpu.sync_copy(cbuf, c_sp.at[pl.ds(base, _ACHUNK)])
        return carry

    lax.fori_loop(0, rows_a // _ACHUNK, chunk_a, 0)
    plsc.subcore_barrier()

    w = sid * _NC + cid

    def start_i(i, b):
        pltpu.async_copy(gly.at[pl.ds(w * rw + i * cb, cb)], ibs[b], sis[b])

    def wait_i(b):
        pltpu.make_async_copy(
            gly.at[pl.ds(w * rw, cb)], ibs[b], sis[b]).wait()

    def start_g(b):
        pltpu.async_copy(c_sp.at[ibs[b]], obs[b], sgs[b])

    def wait_g(b):
        pltpu.make_async_copy(c_sp.at[ibs[b]], obs[b], sgs[b]).wait()

    def start_w(i, b):
        pltpu.async_copy(obs[b], out.at[pl.ds(w * rw + i * cb, cb)], sws[b])

    def wait_w(b):
        pltpu.make_async_copy(
            obs[b], out.at[pl.ds(w * rw, cb)], sws[b]).wait()

    for b in range(_NBUF):
        start_i(b, b)
    for b in range(_NBUF):
        wait_i(b)
        start_g(b)

    def ring(k, carry):
        for b in range(_NBUF):
            i = k * _NBUF + b
            wait_g(b)
            start_w(i, b)

            @pl.when(k < it // _NBUF - 1)
            def _():
                start_i(i + _NBUF, b)
                wait_w(b)
                wait_i(b)
                start_g(b)

        return carry

    lax.fori_loop(0, it // _NBUF, ring, 0)
    for b in range(_NBUF):
        wait_w(b)


@functools.lru_cache(maxsize=None)
def _make_sc_call(n_rows):
    assert n_rows % _NW == 0
    rw = n_rows // _NW
    cb = next(d for d in range(128, 0, -8)
              if rw % d == 0 and (rw // d) % _NBUF == 0)
    it = rw // cb
    mesh = plsc.VectorSubcoreMesh(
        core_axis_name="c", subcore_axis_name="s",
        num_cores=_NC, num_subcores=_NS)
    call = pl.kernel(
        functools.partial(_sc_body, it, cb),
        out_type=jax.ShapeDtypeStruct((n_rows, _D), jnp.float32),
        mesh=mesh,
        scratch_types=[
            pltpu.VMEM((_ACHUNK,), jnp.int32),
            pltpu.VMEM((_ACHUNK,), jnp.int32),
            pltpu.VMEM((_ACHUNK, _D), jnp.float32),
            pltpu.VMEM_SHARED((_GP, _D), jnp.float32),
            pltpu.VMEM((cb,), jnp.int32),
            pltpu.VMEM((cb,), jnp.int32),
            pltpu.VMEM((cb,), jnp.int32),
            pltpu.VMEM((cb, _D), jnp.float32),
            pltpu.VMEM((cb, _D), jnp.float32),
            pltpu.VMEM((cb, _D), jnp.float32),
            pltpu.SemaphoreType.DMA,
            pltpu.SemaphoreType.DMA,
            pltpu.SemaphoreType.DMA,
            pltpu.SemaphoreType.DMA,
            pltpu.SemaphoreType.DMA,
            pltpu.SemaphoreType.DMA,
            pltpu.SemaphoreType.DMA,
            pltpu.SemaphoreType.DMA,
            pltpu.SemaphoreType.DMA,
        ],
    )
    return call


def kernel(glyphs, gl_lookup, entity_table, group_table):
    ent_rn, grp_sh = _renorm(
        jnp.pad(entity_table,
                ((0, _EP - (_MAX_ENTITY + 1)), (0, _D - _ENT_DIM))),
        jnp.pad(group_table,
                ((0, _GTR - (_GROUP_MAX + 1)), (0, _D - _GRP_DIM))),
    )
    gl_grp = jnp.pad(gl_lookup[:, 0], (0, _GP - _NUM_GLYPHS))
    gl_ent = jnp.pad(gl_lookup[:, 1], (0, _GP - _NUM_GLYPHS))
    b, r, c = glyphs.shape
    gly = glyphs.transpose(1, 2, 0).reshape(-1)
    out = _make_sc_call(gly.shape[0])(ent_rn, grp_sh, gl_ent, gl_grp, gly)
    return out.reshape(r, c, b, _D).transpose(2, 0, 1, 3)
